# baseline (speedup 1.0000x reference)
import functools

import jax
import jax.numpy as jnp
from jax import lax
from jax.experimental import pallas as pl
from jax.experimental.pallas import tpu as pltpu
from jax.experimental.pallas import tpu_sc as plsc

SEQ = 8192
BATCH = 4
DIM = 768
RED = SEQ // 2
NCORES = 2
NSUB = 16
LANES = 16

NCH = DIM // 128
RADIX = 256
DMASK = RADIX - 1
NPASS = 4
NSUBCH = 4
CHUNK = SEQ // LANES // NSUBCH
KPAD = SEQ + SEQ // 512


_SBLK = 1024


def _scores_body(x_ref, w_ref, o_ref):
    wb = jnp.broadcast_to(
        w_ref[...].astype(jnp.bfloat16).reshape(DIM, 1), (DIM, 8))
    for b in range(BATCH):
        xb = x_ref[:, b, :].astype(jnp.bfloat16)
        acc = lax.dot_general(
            xb, wb, (((1,), (0,)), ((), ())),
            preferred_element_type=jnp.float32)
        o_ref[b, :] = acc[:, 0]


def _scores_tc(x, w2):
    return pl.pallas_call(
        _scores_body,
        grid=(SEQ // _SBLK,),
        in_specs=[
            pl.BlockSpec((_SBLK, BATCH, DIM), lambda i: (i, 0, 0)),
            pl.BlockSpec((1, DIM), lambda i: (0, 0)),
        ],
        out_specs=pl.BlockSpec((BATCH, _SBLK), lambda i: (0, i)),
        out_shape=jax.ShapeDtypeStruct((BATCH, SEQ), jnp.float32),
    )(x, w2)


def _sc_body(scores_hbm, x_hbm, idx_hbm, w_hbm, xds_hbm,
             score_v, key_a, pos_a, key_b, pos_b,
             h0, h1, h2, h3, rank_v,
             dclip_v, kept_i, kept_w, ids_v,
             gbuf_a, gbuf_b, gbuf_c, gbuf_d,
             isem_a, isem_b, isem_c, isem_d,
             osem_a, osem_b, osem_c, osem_d):
    hists = (h0, h1, h2, h3)
    cidx = lax.axis_index("c")
    sidx = lax.axis_index("s")
    lane = lax.iota(jnp.int32, LANES)

    @pl.when(sidx < 2)
    def _sort_phase():
        b = 2 * cidx + sidx

        with jax.named_scope("sc_extract"):
            pltpu.sync_copy(scores_hbm.at[b], score_v)

        @jax.named_scope("sc_keys")
        def _mk(i, c):
            for u in range(4):
                off = (i * 4 + u) * LANES
                s = score_v[pl.ds(off, LANES)]
                bits = lax.bitcast_convert_type(s, jnp.int32)
                asr = lax.shift_right_arithmetic(bits, 31)
                key = bits ^ (jnp.bitwise_not(asr) & jnp.int32(0x7FFFFFFF))
                q = off + lane
                addr = q + lax.shift_right_logical(q, 9)
                plsc.store_scatter(key_a, [addr], key)
                plsc.store_scatter(pos_a, [addr], q)
            return c
        lax.fori_loop(0, SEQ // LANES // 4, _mk, 0)

        for p in range(NPASS):
            src_k, src_p = (key_a, pos_a) if p % 2 == 0 else (key_b, pos_b)
            dst_k, dst_p = (key_b, pos_b) if p % 2 == 0 else (key_a, pos_a)
            shift = 8 * p

            def _hz(j, c):
                for hh in hists:
                    hh[pl.ds(j * LANES, LANES)] = jnp.zeros((LANES,),
                                                            jnp.int32)
                return c
            lax.fori_loop(0, RADIX, _hz, 0)

            def _h1(i, c):
                for tt in range(2):
                    for s4 in range(NSUBCH):
                        pp = lane * 513 + s4 * CHUNK + i * 2 + tt
                        k = plsc.load_gather(src_k, [pp])
                        dg = lax.shift_right_logical(k, shift) & DMASK
                        plsc.addupdate_scatter(
                            hists[s4], [dg * LANES + lane],
                            jnp.ones((LANES,), jnp.int32))
                return c
            with jax.named_scope("sc_hist"):
                lax.fori_loop(0, CHUNK // 2, _h1, 0)

            def _sc(j, carry):
                sl = pl.ds(j * LANES, LANES)
                vs = [hh[sl] for hh in hists]
                lt = vs[0]
                for v in vs[1:]:
                    lt = lt + v
                cs = plsc.cumsum(lt)
                e = carry + cs - lt
                for s4 in range(NSUBCH):
                    hists[s4][sl] = e
                    e = e + vs[s4]
                return carry + cs[15]
            with jax.named_scope("sc_scan"):
                lax.fori_loop(0, RADIX, _sc, jnp.int32(0))

            last = (p == NPASS - 1)

            def _p1(i, c):
                for tt in range(2):
                    for s4 in range(NSUBCH):
                        pp = lane * 513 + s4 * CHUNK + i * 2 + tt
                        k = plsc.load_gather(src_k, [pp])
                        pv = plsc.load_gather(src_p, [pp])
                        dg = lax.shift_right_logical(k, shift) & DMASK
                        slot = dg * LANES + lane
                        off = plsc.load_gather(hists[s4], [slot])
                        if last:
                            oaddr = off
                        else:
                            oaddr = off + lax.shift_right_logical(off, 9)
                        plsc.store_scatter(dst_k, [oaddr], k)
                        plsc.store_scatter(dst_p, [oaddr], pv)
                        plsc.store_scatter(hists[s4], [slot], off + 1)
                return c
            with jax.named_scope("sc_permute"):
                lax.fori_loop(0, CHUNK // 2, _p1, 0)

        def _rk(k, c):
            for u in range(4):
                off = (k * 4 + u) * LANES
                pv = pos_a[pl.ds(off, LANES)]
                plsc.store_scatter(rank_v, [pv], off + lane)
            return c
        with jax.named_scope("sc_rank"):
            lax.fori_loop(0, SEQ // LANES // 4, _rk, 0)

        def _dc(k, c):
            for u in range(4):
                off = (k * 4 + u) * LANES
                pv = pos_a[pl.ds(RED + off, LANES)]
                s = plsc.load_gather(score_v, [pv])
                dclip_v[pl.ds(off, LANES)] = jnp.clip(s, 0.0, 1.0)
            return c
        with jax.named_scope("sc_dclip"):
            lax.fori_loop(0, RED // LANES // 4, _dc, 0)

        def _cp(i, off):
            o = off
            for u in range(4):
                base = (i * 4 + u) * LANES
                r = rank_v[pl.ds(base, LANES)]
                msk = r < RED
                s = score_v[pl.ds(base, LANES)]
                dval = plsc.load_gather(dclip_v, [r & (RED - 1)])
                w = jnp.clip(s, 0.0, 1.0) - dval
                mi = msk.astype(jnp.int32)
                cs = plsc.cumsum(mi)
                posn = o + cs - mi
                plsc.store_scatter(kept_i, [posn], base + lane, mask=msk)
                plsc.store_scatter(kept_w, [posn], w, mask=msk)
                o = o + cs[15]
            return o
        with jax.named_scope("sc_compact"):
            lax.fori_loop(0, SEQ // LANES // 4, _cp, jnp.int32(0))

        pltpu.sync_copy(kept_i, idx_hbm.at[b])
        pltpu.sync_copy(kept_w, w_hbm.at[b])

    plsc.subcore_barrier()

    gather_scope = jax.named_scope("sc_gather")
    gather_scope.__enter__()
    gb = 2 * cidx + jnp.where(sidx >= 8, 1, 0)
    jbase = (sidx & 7) * 512
    pltpu.sync_copy(idx_hbm.at[gb, pl.ds(jbase, 512)], ids_v)

    def _cv(i, c):
        v = ids_v[pl.ds(i * LANES, LANES)]
        ids_v[pl.ds(i * LANES, LANES)] = v * BATCH + gb
        return c
    lax.fori_loop(0, 512 // LANES, _cv, 0)

    bufs = (gbuf_a, gbuf_b, gbuf_c, gbuf_d)
    isems = (isem_a, isem_b, isem_c, isem_d)
    osems = (osem_a, osem_b, osem_c, osem_d)

    def _src(w):
        return x_hbm.at[ids_v.at[pl.ds(w * 16, 16)]]

    def _dst(w):
        return xds_hbm.at[pl.ds(jbase + w * 16, 16), gb]

    nstep = 32
    nbuf = 4
    inflight = [pltpu.async_copy(_src(k), bufs[k], isems[k])
                for k in range(nbuf)]
    outflight = [None] * nbuf
    for step in range(nstep):
        par = step % nbuf
        inflight[par].wait()
        outflight[par] = pltpu.async_copy(bufs[par], _dst(step), osems[par])
        if step + nbuf < nstep:
            outflight[par].wait()
            inflight[par] = pltpu.async_copy(_src(step + nbuf), bufs[par],
                                             isems[par])
    for k in range(nbuf):
        outflight[(nstep - nbuf + k) % nbuf].wait()
    gather_scope.__exit__(None, None, None)


def _sc_call(scores_sb, x_flat):
    mesh = plsc.VectorSubcoreMesh(
        core_axis_name="c", subcore_axis_name="s",
        num_cores=NCORES, num_subcores=NSUB)
    return pl.kernel(
        _sc_body,
        out_type=(
            jax.ShapeDtypeStruct((BATCH, RED), jnp.int32),
            jax.ShapeDtypeStruct((BATCH, RED), jnp.float32),
            jax.ShapeDtypeStruct((RED, BATCH, DIM), jnp.float32),
        ),
        mesh=mesh,
        compiler_params=pltpu.CompilerParams(needs_layout_passes=False),
        scratch_types=[
            pltpu.VMEM((SEQ,), jnp.float32),
            pltpu.VMEM((KPAD,), jnp.int32),
            pltpu.VMEM((KPAD,), jnp.int32),
            pltpu.VMEM((KPAD,), jnp.int32),
            pltpu.VMEM((KPAD,), jnp.int32),
            pltpu.VMEM((RADIX * LANES,), jnp.int32),
            pltpu.VMEM((RADIX * LANES,), jnp.int32),
            pltpu.VMEM((RADIX * LANES,), jnp.int32),
            pltpu.VMEM((RADIX * LANES,), jnp.int32),
            pltpu.VMEM((SEQ,), jnp.int32),
            pltpu.VMEM((RED,), jnp.float32),
            pltpu.VMEM((RED,), jnp.int32),
            pltpu.VMEM((RED,), jnp.float32),
            pltpu.VMEM((512,), jnp.int32),
            pltpu.VMEM((16, DIM), jnp.float32),
            pltpu.VMEM((16, DIM), jnp.float32),
            pltpu.VMEM((16, DIM), jnp.float32),
            pltpu.VMEM((16, DIM), jnp.float32),
            pltpu.SemaphoreType.DMA,
            pltpu.SemaphoreType.DMA,
            pltpu.SemaphoreType.DMA,
            pltpu.SemaphoreType.DMA,
            pltpu.SemaphoreType.DMA,
            pltpu.SemaphoreType.DMA,
            pltpu.SemaphoreType.DMA,
            pltpu.SemaphoreType.DMA,
        ],
    )(scores_sb, x_flat)


def kernel(x, W):
    scores = _scores_tc(x, W.reshape(1, DIM))
    x_flat = x.reshape(SEQ * BATCH, DIM)
    indexes, weights, xds = _sc_call(scores, x_flat)
    return indexes, weights, xds


# speedup vs baseline: 1.2765x; 1.2765x over previous
import functools

import jax
import jax.numpy as jnp
from jax import lax
from jax.experimental import pallas as pl
from jax.experimental.pallas import tpu as pltpu
from jax.experimental.pallas import tpu_sc as plsc

SEQ = 8192
BATCH = 4
DIM = 768
RED = SEQ // 2
NCORES = 2
NSUB = 16
LANES = 16

NCH = DIM // 128
RADIX = 256
DMASK = RADIX - 1
NPASS = 4
NSUBCH = 4
CHUNK = SEQ // LANES // NSUBCH
KPAD = SEQ + SEQ // 512


_MBLK = 4096


def _scores_body(x_ref, w_ref, o_ref):
    wb = jnp.broadcast_to(
        w_ref[...].astype(jnp.bfloat16).reshape(DIM, 1), (DIM, 8))
    acc = lax.dot_general(
        x_ref[...].astype(jnp.bfloat16), wb, (((1,), (0,)), ((), ())),
        preferred_element_type=jnp.float32)
    o_ref[...] = acc[:, 0].reshape(_MBLK // 128, 128)


def _scores_tc(x_flat, w2):
    return pl.pallas_call(
        _scores_body,
        grid=(SEQ * BATCH // _MBLK,),
        in_specs=[
            pl.BlockSpec((_MBLK, DIM), lambda i: (i, 0)),
            pl.BlockSpec((1, DIM), lambda i: (0, 0)),
        ],
        out_specs=pl.BlockSpec((_MBLK // 128, 128), lambda i: (i, 0)),
        out_shape=jax.ShapeDtypeStruct((SEQ * BATCH // 128, 128),
                                       jnp.float32),
    )(x_flat, w2)


def _sc_body(scores_hbm, x_hbm, idx_hbm, w_hbm, xds_hbm,
             score_v, key_a, pos_a, key_b, pos_b,
             h0, h1, h2, h3, rank_v,
             dclip_v, kept_i, kept_w, ids_v,
             gbuf_a, gbuf_b, gbuf_c, gbuf_d,
             isem_a, isem_b, isem_c, isem_d,
             osem_a, osem_b, osem_c, osem_d):
    hists = (h0, h1, h2, h3)
    cidx = lax.axis_index("c")
    sidx = lax.axis_index("s")
    lane = lax.iota(jnp.int32, LANES)

    @pl.when(sidx < 2)
    def _sort_phase():
        b = 2 * cidx + sidx

        with jax.named_scope("sc_extract"):
          for kq in range(4):
            pltpu.sync_copy(scores_hbm.at[pl.ds(kq * 8192, 8192)],
                            key_b.at[pl.ds(0, 8192)])

            def _ex(i, c):
                q = i * LANES + lane
                bits = plsc.load_gather(key_b, [q * BATCH + b])
                score_v[pl.ds(kq * 2048 + i * LANES, LANES)] = (
                    lax.bitcast_convert_type(bits, jnp.float32))
                return c
            lax.fori_loop(0, 2048 // LANES, _ex, 0)

        @jax.named_scope("sc_keys")
        def _mk(i, c):
            for u in range(4):
                off = (i * 4 + u) * LANES
                s = score_v[pl.ds(off, LANES)]
                bits = lax.bitcast_convert_type(s, jnp.int32)
                asr = lax.shift_right_arithmetic(bits, 31)
                key = bits ^ (jnp.bitwise_not(asr) & jnp.int32(0x7FFFFFFF))
                q = off + lane
                addr = q + lax.shift_right_logical(q, 9)
                plsc.store_scatter(key_a, [addr], key)
                plsc.store_scatter(pos_a, [addr], q)
            return c
        lax.fori_loop(0, SEQ // LANES // 4, _mk, 0)

        for p in range(NPASS):
            src_k, src_p = (key_a, pos_a) if p % 2 == 0 else (key_b, pos_b)
            dst_k, dst_p = (key_b, pos_b) if p % 2 == 0 else (key_a, pos_a)
            shift = 8 * p

            def _hz(j, c):
                for hh in hists:
                    hh[pl.ds(j * LANES, LANES)] = jnp.zeros((LANES,),
                                                            jnp.int32)
                return c
            lax.fori_loop(0, RADIX, _hz, 0)

            def _h1(t, c):
                for s4 in range(NSUBCH):
                    pp = lane * 513 + s4 * CHUNK + t
                    k = plsc.load_gather(src_k, [pp])
                    dg = lax.shift_right_logical(k, shift) & DMASK
                    plsc.addupdate_scatter(hists[s4], [dg * LANES + lane],
                                           jnp.ones((LANES,), jnp.int32))
                return c
            with jax.named_scope("sc_hist"):
                lax.fori_loop(0, CHUNK, _h1, 0)

            def _sc(j, carry):
                sl = pl.ds(j * LANES, LANES)
                vs = [hh[sl] for hh in hists]
                lt = vs[0]
                for v in vs[1:]:
                    lt = lt + v
                cs = plsc.cumsum(lt)
                e = carry + cs - lt
                for s4 in range(NSUBCH):
                    hists[s4][sl] = e
                    e = e + vs[s4]
                return carry + cs[15]
            with jax.named_scope("sc_scan"):
                lax.fori_loop(0, RADIX, _sc, jnp.int32(0))

            last = (p == NPASS - 1)

            def _p1(t, c):
                for s4 in range(NSUBCH):
                    pp = lane * 513 + s4 * CHUNK + t
                    k = plsc.load_gather(src_k, [pp])
                    pv = plsc.load_gather(src_p, [pp])
                    dg = lax.shift_right_logical(k, shift) & DMASK
                    slot = dg * LANES + lane
                    off = plsc.load_gather(hists[s4], [slot])
                    if last:
                        oaddr = off
                    else:
                        oaddr = off + lax.shift_right_logical(off, 9)
                    plsc.store_scatter(dst_k, [oaddr], k)
                    plsc.store_scatter(dst_p, [oaddr], pv)
                    plsc.store_scatter(hists[s4], [slot], off + 1)
                return c
            with jax.named_scope("sc_permute"):
                lax.fori_loop(0, CHUNK, _p1, 0)

        def _rk(k, c):
            for u in range(4):
                off = (k * 4 + u) * LANES
                pv = pos_a[pl.ds(off, LANES)]
                plsc.store_scatter(rank_v, [pv], off + lane)
            return c
        with jax.named_scope("sc_rank"):
            lax.fori_loop(0, SEQ // LANES // 4, _rk, 0)

        def _dc(k, c):
            for u in range(4):
                off = (k * 4 + u) * LANES
                pv = pos_a[pl.ds(RED + off, LANES)]
                s = plsc.load_gather(score_v, [pv])
                dclip_v[pl.ds(off, LANES)] = jnp.clip(s, 0.0, 1.0)
            return c
        with jax.named_scope("sc_dclip"):
            lax.fori_loop(0, RED // LANES // 4, _dc, 0)

        def _cp(i, off):
            o = off
            for u in range(4):
                base = (i * 4 + u) * LANES
                r = rank_v[pl.ds(base, LANES)]
                msk = r < RED
                s = score_v[pl.ds(base, LANES)]
                dval = plsc.load_gather(dclip_v, [r & (RED - 1)])
                w = jnp.clip(s, 0.0, 1.0) - dval
                mi = msk.astype(jnp.int32)
                cs = plsc.cumsum(mi)
                posn = o + cs - mi
                plsc.store_scatter(kept_i, [posn], base + lane, mask=msk)
                plsc.store_scatter(kept_w, [posn], w, mask=msk)
                o = o + cs[15]
            return o
        with jax.named_scope("sc_compact"):
            lax.fori_loop(0, SEQ // LANES // 4, _cp, jnp.int32(0))

        pltpu.sync_copy(kept_i, idx_hbm.at[b])
        pltpu.sync_copy(kept_w, w_hbm.at[b])

    plsc.subcore_barrier()

    gather_scope = jax.named_scope("sc_gather")
    gather_scope.__enter__()
    gb = 2 * cidx + jnp.where(sidx >= 8, 1, 0)
    jbase = (sidx & 7) * 512
    pltpu.sync_copy(idx_hbm.at[gb, pl.ds(jbase, 512)], ids_v)

    def _cv(i, c):
        v = ids_v[pl.ds(i * LANES, LANES)]
        ids_v[pl.ds(i * LANES, LANES)] = v * BATCH + gb
        return c
    lax.fori_loop(0, 512 // LANES, _cv, 0)

    bufs = (gbuf_a, gbuf_b, gbuf_c, gbuf_d)
    isems = (isem_a, isem_b, isem_c, isem_d)
    osems = (osem_a, osem_b, osem_c, osem_d)

    def _src(w):
        return x_hbm.at[ids_v.at[pl.ds(w * 16, 16)]]

    def _dst(w):
        return xds_hbm.at[pl.ds(jbase + w * 16, 16), gb]

    nstep = 32
    nbuf = 4
    inflight = [pltpu.async_copy(_src(k), bufs[k], isems[k])
                for k in range(nbuf)]
    outflight = [None] * nbuf
    for step in range(nstep):
        par = step % nbuf
        inflight[par].wait()
        outflight[par] = pltpu.async_copy(bufs[par], _dst(step), osems[par])
        if step + nbuf < nstep:
            outflight[par].wait()
            inflight[par] = pltpu.async_copy(_src(step + nbuf), bufs[par],
                                             isems[par])
    for k in range(nbuf):
        outflight[(nstep - nbuf + k) % nbuf].wait()
    gather_scope.__exit__(None, None, None)


def _sc_call(scores_i, x_flat):
    mesh = plsc.VectorSubcoreMesh(
        core_axis_name="c", subcore_axis_name="s",
        num_cores=NCORES, num_subcores=NSUB)
    return pl.kernel(
        _sc_body,
        out_type=(
            jax.ShapeDtypeStruct((BATCH, RED), jnp.int32),
            jax.ShapeDtypeStruct((BATCH, RED), jnp.float32),
            jax.ShapeDtypeStruct((RED, BATCH, DIM), jnp.float32),
        ),
        mesh=mesh,
        compiler_params=pltpu.CompilerParams(needs_layout_passes=False),
        scratch_types=[
            pltpu.VMEM((SEQ,), jnp.float32),
            pltpu.VMEM((KPAD,), jnp.int32),
            pltpu.VMEM((KPAD,), jnp.int32),
            pltpu.VMEM((KPAD,), jnp.int32),
            pltpu.VMEM((KPAD,), jnp.int32),
            pltpu.VMEM((RADIX * LANES,), jnp.int32),
            pltpu.VMEM((RADIX * LANES,), jnp.int32),
            pltpu.VMEM((RADIX * LANES,), jnp.int32),
            pltpu.VMEM((RADIX * LANES,), jnp.int32),
            pltpu.VMEM((SEQ,), jnp.int32),
            pltpu.VMEM((RED,), jnp.float32),
            pltpu.VMEM((RED,), jnp.int32),
            pltpu.VMEM((RED,), jnp.float32),
            pltpu.VMEM((512,), jnp.int32),
            pltpu.VMEM((16, DIM), jnp.float32),
            pltpu.VMEM((16, DIM), jnp.float32),
            pltpu.VMEM((16, DIM), jnp.float32),
            pltpu.VMEM((16, DIM), jnp.float32),
            pltpu.SemaphoreType.DMA,
            pltpu.SemaphoreType.DMA,
            pltpu.SemaphoreType.DMA,
            pltpu.SemaphoreType.DMA,
            pltpu.SemaphoreType.DMA,
            pltpu.SemaphoreType.DMA,
            pltpu.SemaphoreType.DMA,
            pltpu.SemaphoreType.DMA,
        ],
    )(scores_i, x_flat)


def kernel(x, W):
    x_flat = x.reshape(SEQ * BATCH, DIM)
    scores = _scores_tc(x_flat, W.reshape(1, DIM))
    scores_i = lax.bitcast_convert_type(scores, jnp.int32).reshape(SEQ * BATCH)
    indexes, weights, xds = _sc_call(scores_i, x_flat)
    return indexes, weights, xds


# speedup vs baseline: 1.2767x; 1.0002x over previous
import functools

import jax
import jax.numpy as jnp
from jax import lax
from jax.experimental import pallas as pl
from jax.experimental.pallas import tpu as pltpu
from jax.experimental.pallas import tpu_sc as plsc

SEQ = 8192
BATCH = 4
DIM = 768
RED = SEQ // 2
NCORES = 2
NSUB = 16
LANES = 16

NCH = DIM // 128
RADIX = 256
DMASK = RADIX - 1
NPASS = 4
NSUBCH = 4
CHUNK = SEQ // LANES // NSUBCH
KPAD = SEQ + SEQ // 512


_MBLK = 8192


def _scores_body(x_ref, w_ref, o_ref):
    wb = jnp.broadcast_to(
        w_ref[...].astype(jnp.bfloat16).reshape(DIM, 1), (DIM, 8))
    acc = lax.dot_general(
        x_ref[...].astype(jnp.bfloat16), wb, (((1,), (0,)), ((), ())),
        preferred_element_type=jnp.float32)
    o_ref[...] = acc[:, 0].reshape(_MBLK // 128, 128)


def _scores_tc(x_flat, w2):
    return pl.pallas_call(
        _scores_body,
        grid=(SEQ * BATCH // _MBLK,),
        in_specs=[
            pl.BlockSpec((_MBLK, DIM), lambda i: (i, 0)),
            pl.BlockSpec((1, DIM), lambda i: (0, 0)),
        ],
        out_specs=pl.BlockSpec((_MBLK // 128, 128), lambda i: (i, 0)),
        out_shape=jax.ShapeDtypeStruct((SEQ * BATCH // 128, 128),
                                       jnp.float32),
    )(x_flat, w2)


def _sc_body(scores_hbm, x_hbm, idx_hbm, w_hbm, xds_hbm,
             score_v, key_a, pos_a, key_b, pos_b,
             h0, h1, h2, h3, rank_v,
             dclip_v, kept_i, kept_w, ids_v,
             gbuf_a, gbuf_b, gbuf_c, gbuf_d,
             isem_a, isem_b, isem_c, isem_d,
             osem_a, osem_b, osem_c, osem_d):
    hists = (h0, h1, h2, h3)
    cidx = lax.axis_index("c")
    sidx = lax.axis_index("s")
    lane = lax.iota(jnp.int32, LANES)

    @pl.when(sidx < 2)
    def _sort_phase():
        b = 2 * cidx + sidx

        with jax.named_scope("sc_extract"):
          for kq in range(4):
            pltpu.sync_copy(scores_hbm.at[pl.ds(kq * 8192, 8192)],
                            key_b.at[pl.ds(0, 8192)])

            def _ex(i, c):
                q = i * LANES + lane
                bits = plsc.load_gather(key_b, [q * BATCH + b])
                score_v[pl.ds(kq * 2048 + i * LANES, LANES)] = (
                    lax.bitcast_convert_type(bits, jnp.float32))
                return c
            lax.fori_loop(0, 2048 // LANES, _ex, 0)

        @jax.named_scope("sc_keys")
        def _mk(i, c):
            for u in range(4):
                off = (i * 4 + u) * LANES
                s = score_v[pl.ds(off, LANES)]
                bits = lax.bitcast_convert_type(s, jnp.int32)
                asr = lax.shift_right_arithmetic(bits, 31)
                key = bits ^ (jnp.bitwise_not(asr) & jnp.int32(0x7FFFFFFF))
                q = off + lane
                addr = q + lax.shift_right_logical(q, 9)
                plsc.store_scatter(key_a, [addr], key)
                plsc.store_scatter(pos_a, [addr], q)
            return c
        lax.fori_loop(0, SEQ // LANES // 4, _mk, 0)

        for p in range(NPASS):
            src_k, src_p = (key_a, pos_a) if p % 2 == 0 else (key_b, pos_b)
            dst_k, dst_p = (key_b, pos_b) if p % 2 == 0 else (key_a, pos_a)
            shift = 8 * p

            def _hz(j, c):
                for hh in hists:
                    hh[pl.ds(j * LANES, LANES)] = jnp.zeros((LANES,),
                                                            jnp.int32)
                return c
            lax.fori_loop(0, RADIX, _hz, 0)

            def _h1(i, c):
                for tt in range(2):
                    for s4 in range(NSUBCH):
                        pp = lane * 513 + s4 * CHUNK + i * 2 + tt
                        k = plsc.load_gather(src_k, [pp])
                        dg = lax.shift_right_logical(k, shift) & DMASK
                        plsc.addupdate_scatter(
                            hists[s4], [dg * LANES + lane],
                            jnp.ones((LANES,), jnp.int32))
                return c
            with jax.named_scope("sc_hist"):
                lax.fori_loop(0, CHUNK // 2, _h1, 0)

            def _sc(j, carry):
                sl = pl.ds(j * LANES, LANES)
                vs = [hh[sl] for hh in hists]
                lt = vs[0]
                for v in vs[1:]:
                    lt = lt + v
                cs = plsc.cumsum(lt)
                e = carry + cs - lt
                for s4 in range(NSUBCH):
                    hists[s4][sl] = e
                    e = e + vs[s4]
                return carry + cs[15]
            with jax.named_scope("sc_scan"):
                lax.fori_loop(0, RADIX, _sc, jnp.int32(0))

            last = (p == NPASS - 1)

            def _p1(i, c):
                for tt in range(2):
                    for s4 in range(NSUBCH):
                        pp = lane * 513 + s4 * CHUNK + i * 2 + tt
                        k = plsc.load_gather(src_k, [pp])
                        pv = plsc.load_gather(src_p, [pp])
                        dg = lax.shift_right_logical(k, shift) & DMASK
                        slot = dg * LANES + lane
                        off = plsc.load_gather(hists[s4], [slot])
                        if last:
                            oaddr = off
                        else:
                            oaddr = off + lax.shift_right_logical(off, 9)
                        plsc.store_scatter(dst_k, [oaddr], k)
                        plsc.store_scatter(dst_p, [oaddr], pv)
                        plsc.store_scatter(hists[s4], [slot], off + 1)
                return c
            with jax.named_scope("sc_permute"):
                lax.fori_loop(0, CHUNK // 2, _p1, 0)

        def _rk(k, c):
            for u in range(4):
                off = (k * 4 + u) * LANES
                pv = pos_a[pl.ds(off, LANES)]
                plsc.store_scatter(rank_v, [pv], off + lane)
            return c
        with jax.named_scope("sc_rank"):
            lax.fori_loop(0, SEQ // LANES // 4, _rk, 0)

        def _dc(k, c):
            for u in range(4):
                off = (k * 4 + u) * LANES
                pv = pos_a[pl.ds(RED + off, LANES)]
                s = plsc.load_gather(score_v, [pv])
                dclip_v[pl.ds(off, LANES)] = jnp.clip(s, 0.0, 1.0)
            return c
        with jax.named_scope("sc_dclip"):
            lax.fori_loop(0, RED // LANES // 4, _dc, 0)

        def _cp(i, off):
            o = off
            for u in range(4):
                base = (i * 4 + u) * LANES
                r = rank_v[pl.ds(base, LANES)]
                msk = r < RED
                s = score_v[pl.ds(base, LANES)]
                dval = plsc.load_gather(dclip_v, [r & (RED - 1)])
                w = jnp.clip(s, 0.0, 1.0) - dval
                mi = msk.astype(jnp.int32)
                cs = plsc.cumsum(mi)
                posn = o + cs - mi
                plsc.store_scatter(kept_i, [posn], base + lane, mask=msk)
                plsc.store_scatter(kept_w, [posn], w, mask=msk)
                o = o + cs[15]
            return o
        with jax.named_scope("sc_compact"):
            lax.fori_loop(0, SEQ // LANES // 4, _cp, jnp.int32(0))

        pltpu.sync_copy(kept_i, idx_hbm.at[b])
        pltpu.sync_copy(kept_w, w_hbm.at[b])

    plsc.subcore_barrier()

    gather_scope = jax.named_scope("sc_gather")
    gather_scope.__enter__()
    gb = 2 * cidx + jnp.where(sidx >= 8, 1, 0)
    jbase = (sidx & 7) * 512
    pltpu.sync_copy(idx_hbm.at[gb, pl.ds(jbase, 512)], ids_v)

    def _cv(i, c):
        v = ids_v[pl.ds(i * LANES, LANES)]
        ids_v[pl.ds(i * LANES, LANES)] = v * BATCH + gb
        return c
    lax.fori_loop(0, 512 // LANES, _cv, 0)

    bufs = (gbuf_a, gbuf_b, gbuf_c, gbuf_d)
    isems = (isem_a, isem_b, isem_c, isem_d)
    osems = (osem_a, osem_b, osem_c, osem_d)

    def _src(w):
        return x_hbm.at[ids_v.at[pl.ds(w * 16, 16)]]

    def _dst(w):
        return xds_hbm.at[pl.ds(jbase + w * 16, 16), gb]

    nstep = 32
    nbuf = 4
    inflight = [pltpu.async_copy(_src(k), bufs[k], isems[k])
                for k in range(nbuf)]
    outflight = [None] * nbuf
    for step in range(nstep):
        par = step % nbuf
        inflight[par].wait()
        outflight[par] = pltpu.async_copy(bufs[par], _dst(step), osems[par])
        if step + nbuf < nstep:
            outflight[par].wait()
            inflight[par] = pltpu.async_copy(_src(step + nbuf), bufs[par],
                                             isems[par])
    for k in range(nbuf):
        outflight[(nstep - nbuf + k) % nbuf].wait()
    gather_scope.__exit__(None, None, None)


def _sc_call(scores_i, x_flat):
    mesh = plsc.VectorSubcoreMesh(
        core_axis_name="c", subcore_axis_name="s",
        num_cores=NCORES, num_subcores=NSUB)
    return pl.kernel(
        _sc_body,
        out_type=(
            jax.ShapeDtypeStruct((BATCH, RED), jnp.int32),
            jax.ShapeDtypeStruct((BATCH, RED), jnp.float32),
            jax.ShapeDtypeStruct((RED, BATCH, DIM), jnp.float32),
        ),
        mesh=mesh,
        compiler_params=pltpu.CompilerParams(needs_layout_passes=False),
        scratch_types=[
            pltpu.VMEM((SEQ,), jnp.float32),
            pltpu.VMEM((KPAD,), jnp.int32),
            pltpu.VMEM((KPAD,), jnp.int32),
            pltpu.VMEM((KPAD,), jnp.int32),
            pltpu.VMEM((KPAD,), jnp.int32),
            pltpu.VMEM((RADIX * LANES,), jnp.int32),
            pltpu.VMEM((RADIX * LANES,), jnp.int32),
            pltpu.VMEM((RADIX * LANES,), jnp.int32),
            pltpu.VMEM((RADIX * LANES,), jnp.int32),
            pltpu.VMEM((SEQ,), jnp.int32),
            pltpu.VMEM((RED,), jnp.float32),
            pltpu.VMEM((RED,), jnp.int32),
            pltpu.VMEM((RED,), jnp.float32),
            pltpu.VMEM((512,), jnp.int32),
            pltpu.VMEM((16, DIM), jnp.float32),
            pltpu.VMEM((16, DIM), jnp.float32),
            pltpu.VMEM((16, DIM), jnp.float32),
            pltpu.VMEM((16, DIM), jnp.float32),
            pltpu.SemaphoreType.DMA,
            pltpu.SemaphoreType.DMA,
            pltpu.SemaphoreType.DMA,
            pltpu.SemaphoreType.DMA,
            pltpu.SemaphoreType.DMA,
            pltpu.SemaphoreType.DMA,
            pltpu.SemaphoreType.DMA,
            pltpu.SemaphoreType.DMA,
        ],
    )(scores_i, x_flat)


def kernel(x, W):
    x_flat = x.reshape(SEQ * BATCH, DIM)
    scores = _scores_tc(x_flat, W.reshape(1, DIM))
    scores_i = lax.bitcast_convert_type(scores, jnp.int32).reshape(SEQ * BATCH)
    indexes, weights, xds = _sc_call(scores_i, x_flat)
    return indexes, weights, xds
